# baseline (speedup 1.0000x reference)
import functools

import jax
import jax.numpy as jnp
from jax import lax
from jax.experimental import pallas as pl
from jax.experimental.pallas import tpu as pltpu
from jax.experimental.pallas import tpu_sc as plsc

TILE_N = 1024
TILE_N2 = 1024
_PREC = jax.lax.Precision.DEFAULT


def _stage1a(xyz1_ref, xyz2t_ref, fidx_ref, wout_ref):
    b = pl.program_id(0)
    x = xyz1_ref[0]
    a = xyz2t_ref[0]
    S = a.shape[1]

    dx = x[:, 0:1] - a[0:1, :]
    dy = x[:, 1:2] - a[1:2, :]
    dz = x[:, 2:3] - a[2:3, :]
    d = dx * dx + dy * dy + dz * dz

    iota = jax.lax.broadcasted_iota(jnp.int32, d.shape, 1)
    keys = (jax.lax.bitcast_convert_type(d, jnp.int32)
            & jnp.int32(-1024)) | iota

    kmax = jnp.int32(0x7FFFFFFF)
    k_work = keys
    ks = []
    for k in range(3):
        mk = jnp.min(k_work, axis=1, keepdims=True)
        ks.append(mk)
        if k < 2:
            k_work = jnp.where(k_work == mk, kmax, k_work)

    ms = [jax.lax.bitcast_convert_type(mk & jnp.int32(-1024), jnp.float32)
          for mk in ks]
    r1 = 1.0 / (ms[0] + 1e-8)
    r2 = 1.0 / (ms[1] + 1e-8)
    r3 = 1.0 / (ms[2] + 1e-8)
    norm = r1 + r2 + r3

    base = b * S
    for k in range(3):
        fidx_ref[0, :, k:k + 1] = (ks[k] & jnp.int32(1023)) + base
    ones16 = jnp.ones((1, 16), jnp.float32)
    wout_ref[0, :, 0:16] = (r1 / norm) * ones16
    wout_ref[0, :, 16:32] = (r2 / norm) * ones16
    wout_ref[0, :, 32:48] = (r3 / norm) * ones16


def _sc_interp(table_ref, idxt_ref, wrep_ref, out_ref,
               i1_v, i2_v, i3_v, w1_v, w2_v, w3_v,
               r1_v, r2_v, r3_v, out_v, sem):
    C = i1_v.shape[0]
    D = table_ref.shape[1]
    nw = 32
    total = out_ref.shape[0]
    per_w = total // nw
    nchunk = per_w // C
    wid = lax.axis_index("s") * 2 + lax.axis_index("c")
    base = wid * per_w

    def chunk(ci, carry):
        off = base + ci * C
        pltpu.sync_copy(idxt_ref.at[0, pl.ds(off, C)], i1_v)
        pltpu.sync_copy(idxt_ref.at[1, pl.ds(off, C)], i2_v)
        pltpu.sync_copy(idxt_ref.at[2, pl.ds(off, C)], i3_v)
        pltpu.sync_copy(wrep_ref.at[0, pl.ds(off, C)], w1_v)
        pltpu.sync_copy(wrep_ref.at[1, pl.ds(off, C)], w2_v)
        pltpu.sync_copy(wrep_ref.at[2, pl.ds(off, C)], w3_v)
        c1 = pltpu.async_copy(table_ref.at[i1_v], r1_v, sem)
        c2 = pltpu.async_copy(table_ref.at[i2_v], r2_v, sem)
        c3 = pltpu.async_copy(table_ref.at[i3_v], r3_v, sem)
        c1.wait()
        c2.wait()
        c3.wait()

        def point(p, carry2):
            w1b = w1_v[p]
            w2b = w2_v[p]
            w3b = w3_v[p]
            for j in range(D // 16):
                sl = pl.ds(j * 16, 16)
                acc = (w1b * r1_v[p, sl] + w2b * r2_v[p, sl]
                       + w3b * r3_v[p, sl])
                out_v[p, sl] = acc
            return carry2

        lax.fori_loop(0, C, point, 0)
        pltpu.sync_copy(out_v, out_ref.at[pl.ds(off, C)])
        return carry

    lax.fori_loop(0, nchunk, chunk, 0)


def _stage1c(p1_ref, interp_ref, w1_ref, b1_ref, x1_ref, sum_ref, sq_ref):
    b = pl.program_id(0)
    n = pl.program_id(1)
    feats = jnp.concatenate([p1_ref[0], interp_ref[0]], axis=1)
    x1 = jax.lax.dot_general(
        feats, w1_ref[...], (((1,), (1,)), ((), ())),
        preferred_element_type=jnp.float32, precision=_PREC)
    x1 = x1 + b1_ref[...]
    x1_ref[0] = x1.astype(jnp.bfloat16)

    @pl.when((b == 0) & (n == 0))
    def _():
        sum_ref[...] = jnp.zeros_like(sum_ref)
        sq_ref[...] = jnp.zeros_like(sq_ref)

    sum_ref[...] += jnp.sum(x1, axis=0, keepdims=True)
    sq_ref[...] += jnp.sum(x1 * x1, axis=0, keepdims=True)


def _stage23(cnt_inv, x1_ref, w2_ref, b2_ref, sc_ref, sh_ref, g2_ref, be2_ref,
             out_ref, y_scr, sum_scr, sq_scr):
    p = pl.program_id(0)
    b = pl.program_id(1)
    n = pl.program_id(2)
    nt = pl.num_programs(2)
    i = b * nt + n

    @pl.when(p == 0)
    def _phase0():
        z = jnp.maximum(x1_ref[0].astype(jnp.float32) * sc_ref[...]
                        + sh_ref[...], 0.0)
        y = jax.lax.dot_general(
            z, w2_ref[...], (((1,), (1,)), ((), ())),
            preferred_element_type=jnp.float32, precision=_PREC)
        y = y + b2_ref[...]
        y_scr[i] = y.astype(jnp.bfloat16)

        @pl.when(i == 0)
        def _():
            sum_scr[...] = jnp.zeros_like(sum_scr)
            sq_scr[...] = jnp.zeros_like(sq_scr)

        sum_scr[...] += jnp.sum(y, axis=0, keepdims=True)
        sq_scr[...] += jnp.sum(y * y, axis=0, keepdims=True)

    @pl.when(p == 1)
    def _phase1():
        mean = sum_scr[...] * cnt_inv
        var = sq_scr[...] * cnt_inv - mean * mean
        scale = g2_ref[...] / jnp.sqrt(var + 1e-5)
        shift = be2_ref[...] - mean * scale
        y = y_scr[i].astype(jnp.float32)
        t = jnp.maximum(y * scale + shift, 0.0)
        out_ref[0] = t.T


def kernel(xyz1, xyz2, points1, points2, W1, b1, g1, be1, W2, b2, g2, be2):
    B, N, _ = xyz1.shape
    S = xyz2.shape[1]
    D1 = points1.shape[2]
    D2 = points2.shape[2]
    C1 = W1.shape[0]
    C2 = W2.shape[0]
    NT = N // TILE_N
    cnt = float(B * N)
    BN = B * N

    xyz2t = jnp.transpose(xyz2, (0, 2, 1))
    b1r = b1.reshape(1, C1)
    b2r = b2.reshape(1, C2)

    fidx, wout = pl.pallas_call(
        _stage1a,
        grid=(B, NT),
        in_specs=[
            pl.BlockSpec((1, TILE_N, 3), lambda b, n: (b, n, 0)),
            pl.BlockSpec((1, 3, S), lambda b, n: (b, 0, 0)),
        ],
        out_specs=[
            pl.BlockSpec((1, TILE_N, 4), lambda b, n: (b, n, 0)),
            pl.BlockSpec((1, TILE_N, 48), lambda b, n: (b, n, 0)),
        ],
        out_shape=[
            jax.ShapeDtypeStruct((B, N, 4), jnp.int32),
            jax.ShapeDtypeStruct((B, N, 48), jnp.float32),
        ],
    )(xyz1, xyz2t)

    idxt = fidx[:, :, :3].transpose(2, 0, 1).reshape(3, BN)
    wrep = wout.reshape(B, N, 3, 16).transpose(2, 0, 1, 3).reshape(3, BN, 16)
    table = points2.reshape(B * S, D2)

    CHUNK = 32
    sc_fn = functools.partial(
        pl.kernel,
        mesh=plsc.VectorSubcoreMesh(core_axis_name="c", subcore_axis_name="s"),
        out_type=jax.ShapeDtypeStruct((BN, D2), jnp.float32),
        scratch_types=[
            pltpu.VMEM((CHUNK,), jnp.int32),
            pltpu.VMEM((CHUNK,), jnp.int32),
            pltpu.VMEM((CHUNK,), jnp.int32),
            pltpu.VMEM((CHUNK, 16), jnp.float32),
            pltpu.VMEM((CHUNK, 16), jnp.float32),
            pltpu.VMEM((CHUNK, 16), jnp.float32),
            pltpu.VMEM((CHUNK, D2), jnp.float32),
            pltpu.VMEM((CHUNK, D2), jnp.float32),
            pltpu.VMEM((CHUNK, D2), jnp.float32),
            pltpu.VMEM((CHUNK, D2), jnp.float32),
            pltpu.SemaphoreType.DMA,
        ],
    )(_sc_interp)
    interp = sc_fn(table, idxt, wrep).reshape(B, N, D2)

    x1, s1, q1 = pl.pallas_call(
        _stage1c,
        grid=(B, NT),
        in_specs=[
            pl.BlockSpec((1, TILE_N, D1), lambda b, n: (b, n, 0)),
            pl.BlockSpec((1, TILE_N, D2), lambda b, n: (b, n, 0)),
            pl.BlockSpec((C1, D1 + D2), lambda b, n: (0, 0)),
            pl.BlockSpec((1, C1), lambda b, n: (0, 0)),
        ],
        out_specs=[
            pl.BlockSpec((1, TILE_N, C1), lambda b, n: (b, n, 0)),
            pl.BlockSpec((1, C1), lambda b, n: (0, 0)),
            pl.BlockSpec((1, C1), lambda b, n: (0, 0)),
        ],
        out_shape=[
            jax.ShapeDtypeStruct((B, N, C1), jnp.bfloat16),
            jax.ShapeDtypeStruct((1, C1), jnp.float32),
            jax.ShapeDtypeStruct((1, C1), jnp.float32),
        ],
    )(points1, interp, W1, b1r)

    mean1 = s1[0] / cnt
    var1 = q1[0] / cnt - mean1 * mean1
    scale1 = g1 / jnp.sqrt(var1 + 1e-5)
    shift1 = be1 - mean1 * scale1

    NT2 = N // TILE_N2
    out = pl.pallas_call(
        functools.partial(_stage23, 1.0 / cnt),
        grid=(2, B, NT2),
        in_specs=[
            pl.BlockSpec((1, TILE_N2, C1),
                         lambda p, b, n: (b * (1 - p), n * (1 - p), 0)),
            pl.BlockSpec((C2, C1), lambda p, b, n: (0, 0)),
            pl.BlockSpec((1, C2), lambda p, b, n: (0, 0)),
            pl.BlockSpec((1, C1), lambda p, b, n: (0, 0)),
            pl.BlockSpec((1, C1), lambda p, b, n: (0, 0)),
            pl.BlockSpec((1, C2), lambda p, b, n: (0, 0)),
            pl.BlockSpec((1, C2), lambda p, b, n: (0, 0)),
        ],
        out_specs=pl.BlockSpec((1, C2, TILE_N2),
                               lambda p, b, n: (b * p, 0, n * p)),
        out_shape=jax.ShapeDtypeStruct((B, C2, N), jnp.float32),
        scratch_shapes=[
            pltpu.VMEM((B * NT2, TILE_N2, C2), jnp.bfloat16),
            pltpu.VMEM((1, C2), jnp.float32),
            pltpu.VMEM((1, C2), jnp.float32),
        ],
    )(x1, W2, b2r, scale1.reshape(1, C1), shift1.reshape(1, C1),
      g2.reshape(1, C2), be2.reshape(1, C2))

    return out


# speedup vs baseline: 2.0954x; 2.0954x over previous
import jax
import jax.numpy as jnp
from jax.experimental import pallas as pl
from jax.experimental.pallas import tpu as pltpu

TILE_N = 1024
TILE_N2 = 1024
_PREC = jax.lax.Precision.DEFAULT


def _stage1(xyz1_ref, xyz2t_ref, p1_ref, p2_ref, w1_ref, b1_ref,
            x1_ref, sum_ref, sq_ref):
    b = pl.program_id(0)
    n = pl.program_id(1)
    x = xyz1_ref[0]
    a = xyz2t_ref[0]

    dx = x[:, 0:1] - a[0:1, :]
    dy = x[:, 1:2] - a[1:2, :]
    dz = x[:, 2:3] - a[2:3, :]
    d = dx * dx + dy * dy + dz * dz

    iota = jax.lax.broadcasted_iota(jnp.int32, d.shape, 1)
    keys = (jax.lax.bitcast_convert_type(d, jnp.int32)
            & jnp.int32(-1024)) | iota

    kmax = jnp.int32(0x7FFFFFFF)
    k_work = keys
    ks = []
    for k in range(3):
        mk = jnp.min(k_work, axis=1, keepdims=True)
        ks.append(mk)
        if k < 2:
            k_work = jnp.where(k_work == mk, kmax, k_work)

    ms = [jax.lax.bitcast_convert_type(mk & jnp.int32(-1024), jnp.float32)
          for mk in ks]
    r1 = 1.0 / (ms[0] + 1e-8)
    r2 = 1.0 / (ms[1] + 1e-8)
    r3 = 1.0 / (ms[2] + 1e-8)
    norm = r1 + r2 + r3
    w1 = r1 / norm
    w2 = r2 / norm
    w3 = r3 / norm

    zero = jnp.float32(0.0)
    wmat = (jnp.where(keys == ks[0], w1, zero)
            + jnp.where(keys == ks[1], w2, zero)
            + jnp.where(keys == ks[2], w3, zero))

    interp = jax.lax.dot_general(
        wmat, p2_ref[0], (((1,), (0,)), ((), ())),
        preferred_element_type=jnp.float32, precision=_PREC)

    feats = jnp.concatenate([p1_ref[0], interp], axis=1)
    x1 = jax.lax.dot_general(
        feats, w1_ref[...], (((1,), (1,)), ((), ())),
        preferred_element_type=jnp.float32, precision=_PREC)
    x1 = x1 + b1_ref[...]
    x1_ref[0] = x1.astype(jnp.bfloat16)

    @pl.when((b == 0) & (n == 0))
    def _():
        sum_ref[...] = jnp.zeros_like(sum_ref)
        sq_ref[...] = jnp.zeros_like(sq_ref)

    sum_ref[...] += jnp.sum(x1, axis=0, keepdims=True)
    sq_ref[...] += jnp.sum(x1 * x1, axis=0, keepdims=True)


def _stage23(cnt_inv, x1_ref, w2_ref, b2_ref, sc_ref, sh_ref, g2_ref, be2_ref,
             out_ref, y_scr, sum_scr, sq_scr):
    p = pl.program_id(0)
    b = pl.program_id(1)
    n = pl.program_id(2)
    nt = pl.num_programs(2)
    i = b * nt + n

    @pl.when(p == 0)
    def _phase0():
        z = jnp.maximum(x1_ref[0].astype(jnp.float32) * sc_ref[...]
                        + sh_ref[...], 0.0)
        y = jax.lax.dot_general(
            z, w2_ref[...], (((1,), (1,)), ((), ())),
            preferred_element_type=jnp.float32, precision=_PREC)
        y = y + b2_ref[...]
        y_scr[i] = y.astype(jnp.bfloat16)

        @pl.when(i == 0)
        def _():
            sum_scr[...] = jnp.zeros_like(sum_scr)
            sq_scr[...] = jnp.zeros_like(sq_scr)

        sum_scr[...] += jnp.sum(y, axis=0, keepdims=True)
        sq_scr[...] += jnp.sum(y * y, axis=0, keepdims=True)

    @pl.when(p == 1)
    def _phase1():
        mean = sum_scr[...] * cnt_inv
        var = sq_scr[...] * cnt_inv - mean * mean
        scale = g2_ref[...] / jnp.sqrt(var + 1e-5)
        shift = be2_ref[...] - mean * scale
        y = y_scr[i].astype(jnp.float32)
        t = jnp.maximum(y * scale + shift, 0.0)
        out_ref[0] = t.T


def kernel(xyz1, xyz2, points1, points2, W1, b1, g1, be1, W2, b2, g2, be2):
    B, N, _ = xyz1.shape
    S = xyz2.shape[1]
    D1 = points1.shape[2]
    D2 = points2.shape[2]
    C1 = W1.shape[0]
    C2 = W2.shape[0]
    NT = N // TILE_N
    cnt = float(B * N)

    xyz2t = jnp.transpose(xyz2, (0, 2, 1))
    b1r = b1.reshape(1, C1)
    b2r = b2.reshape(1, C2)

    x1, s1, q1 = pl.pallas_call(
        _stage1,
        grid=(B, NT),
        in_specs=[
            pl.BlockSpec((1, TILE_N, 3), lambda b, n: (b, n, 0)),
            pl.BlockSpec((1, 3, S), lambda b, n: (b, 0, 0)),
            pl.BlockSpec((1, TILE_N, D1), lambda b, n: (b, n, 0)),
            pl.BlockSpec((1, S, D2), lambda b, n: (b, 0, 0)),
            pl.BlockSpec((C1, D1 + D2), lambda b, n: (0, 0)),
            pl.BlockSpec((1, C1), lambda b, n: (0, 0)),
        ],
        out_specs=[
            pl.BlockSpec((1, TILE_N, C1), lambda b, n: (b, n, 0)),
            pl.BlockSpec((1, C1), lambda b, n: (0, 0)),
            pl.BlockSpec((1, C1), lambda b, n: (0, 0)),
        ],
        out_shape=[
            jax.ShapeDtypeStruct((B, N, C1), jnp.bfloat16),
            jax.ShapeDtypeStruct((1, C1), jnp.float32),
            jax.ShapeDtypeStruct((1, C1), jnp.float32),
        ],
    )(xyz1, xyz2t, points1, points2, W1, b1r)

    mean1 = s1[0] / cnt
    var1 = q1[0] / cnt - mean1 * mean1
    scale1 = g1 / jnp.sqrt(var1 + 1e-5)
    shift1 = be1 - mean1 * scale1

    NT2 = N // TILE_N2
    import functools
    out = pl.pallas_call(
        functools.partial(_stage23, 1.0 / cnt),
        grid=(2, B, NT2),
        in_specs=[
            pl.BlockSpec((1, TILE_N2, C1),
                         lambda p, b, n: (b * (1 - p), n * (1 - p), 0)),
            pl.BlockSpec((C2, C1), lambda p, b, n: (0, 0)),
            pl.BlockSpec((1, C2), lambda p, b, n: (0, 0)),
            pl.BlockSpec((1, C1), lambda p, b, n: (0, 0)),
            pl.BlockSpec((1, C1), lambda p, b, n: (0, 0)),
            pl.BlockSpec((1, C2), lambda p, b, n: (0, 0)),
            pl.BlockSpec((1, C2), lambda p, b, n: (0, 0)),
        ],
        out_specs=pl.BlockSpec((1, C2, TILE_N2),
                               lambda p, b, n: (b * p, 0, n * p)),
        out_shape=jax.ShapeDtypeStruct((B, C2, N), jnp.float32),
        scratch_shapes=[
            pltpu.VMEM((B * NT2, TILE_N2, C2), jnp.bfloat16),
            pltpu.VMEM((1, C2), jnp.float32),
            pltpu.VMEM((1, C2), jnp.float32),
        ],
    )(x1, W2, b2r, scale1.reshape(1, C1), shift1.reshape(1, C1),
      g2.reshape(1, C2), be2.reshape(1, C2))

    return out
